# baseline (speedup 1.0000x reference)
import functools

import jax
import jax.numpy as jnp
from jax import lax
from jax.experimental import pallas as pl
from jax.experimental.pallas import tpu as pltpu
from jax.experimental.pallas import tpu_sc as plsc

B = 16
N = 5000
NP = 5120
NRI = 625
NRO = 640
NCHUNK = 320
K = 100
KPAD = 104
NEG = -3.0e38


GB = 4


def _tc_body(logits_ref, obj_ref, keys_ref, cmax_ref, labels_ref):
    x = logits_ref[...].reshape(GB * NRI, 8, 91)
    xk = x[..., :90]
    m = jnp.max(xk, axis=-1)
    cls_iota = lax.broadcasted_iota(jnp.int32, (GB * NRI, 8, 90), 2)
    lbl = jnp.min(jnp.where(xk == m[..., None], cls_iota, 1000000), axis=-1)
    o = obj_ref[...].reshape(GB * NRO, 8)
    o = jnp.exp(-jnp.concatenate(
        [o[i * NRO:i * NRO + NRI] for i in range(GB)], axis=0))
    p = jax.nn.sigmoid(m)
    sk = o * p
    su = o * (1.0 - p)
    choose = su > sk
    fs = jnp.where(choose, su, sk)
    flb = jnp.where(choose, 90, lbl)
    keep = fs > 0.05
    gidx = ((lax.broadcasted_iota(jnp.int32, (GB * NRI, 8), 0) % NRI) * 8
            + lax.broadcasted_iota(jnp.int32, (GB * NRI, 8), 1))
    enc = -(1e30 + gidx.astype(jnp.float32) * 1e24)
    key = jnp.where(keep, fs, enc)
    pad_flat = (N
                + lax.broadcasted_iota(jnp.int32, (NRO - NRI, 8), 0) * 8
                + lax.broadcasted_iota(jnp.int32, (NRO - NRI, 8), 1))
    pad_enc = -(1e30 + pad_flat.astype(jnp.float32) * 1e24)
    key_full = jnp.concatenate(
        [z for i in range(GB)
         for z in (key[i * NRI:(i + 1) * NRI], pad_enc)], axis=0)
    keys_ref[...] = key_full.reshape(GB, NRO, 8)
    cmax_ref[...] = jnp.max(key_full, axis=1, keepdims=True).reshape(
        GB, NRO, 1)
    lab_full = jnp.concatenate(
        [z for i in range(GB)
         for z in (flb[i * NRI:(i + 1) * NRI],
                   jnp.zeros((NRO - NRI, 8), jnp.int32))], axis=0)
    labels_ref[...] = lab_full.reshape(GB, NRO, 8)


_sc_mesh = plsc.VectorSubcoreMesh(core_axis_name="c", subcore_axis_name="s")


@functools.partial(
    pl.kernel,
    mesh=_sc_mesh,
    out_type=[
        jax.ShapeDtypeStruct((B * KPAD,), jnp.float32),
        jax.ShapeDtypeStruct((B * KPAD,), jnp.int32),
        jax.ShapeDtypeStruct((B * KPAD * 4,), jnp.float32),
    ],
    scratch_types=[
        pltpu.VMEM((NP + 8,), jnp.float32),
        pltpu.VMEM((NRO,), jnp.float32),
        pltpu.VMEM((NP + 8,), jnp.int32),
        pltpu.VMEM((N * 4,), jnp.float32),
        pltpu.VMEM((32,), jnp.float32),
        pltpu.VMEM((KPAD,), jnp.float32),
        pltpu.VMEM((KPAD,), jnp.int32),
        pltpu.VMEM((KPAD * 4,), jnp.float32),
    ],
    compiler_params=pltpu.CompilerParams(needs_layout_passes=False),
)
def _sc_topk(keys_hbm, cm_hbm, labels_hbm, boxes_hbm, ts_hbm,
             so_hbm, lo_hbm, bo_hbm,
             keys_v, cm_v, labels_v, boxes_v, ts_v, os_v, ol_v, ob_v):
    wid = lax.axis_index("s") * 2 + lax.axis_index("c")

    @pl.when(wid < B)
    def _():
        b = wid
        pltpu.sync_copy(keys_hbm.at[pl.ds(b * NP, NP)],
                        keys_v.at[pl.ds(0, NP)])
        pltpu.sync_copy(cm_hbm.at[pl.ds(b * NRO, NRO)], cm_v)
        pltpu.sync_copy(labels_hbm.at[pl.ds(b * NP, NP)],
                        labels_v.at[pl.ds(0, NP)])
        pltpu.sync_copy(boxes_hbm.at[pl.ds(b * N * 4, N * 4)], boxes_v)
        pltpu.sync_copy(ts_hbm, ts_v)
        lanes = lax.iota(jnp.int32, 16)
        hvec = plsc.load_gather(ts_v, [jnp.full((16,), 2 * b, jnp.int32)])
        wvec = plsc.load_gather(ts_v, [jnp.full((16,), 2 * b + 1, jnp.int32)])
        hs = jnp.max(hvec)
        ws = jnp.max(wvec)
        os_v[pl.ds(KPAD - 16, 16)] = jnp.zeros((16,), jnp.float32)
        ol_v[pl.ds(KPAD - 16, 16)] = jnp.zeros((16,), jnp.int32)
        ob_v[pl.ds(KPAD * 4 - 16, 16)] = jnp.zeros((16,), jnp.float32)

        def sel_body(k, carry):
            bestv = jnp.full((16,), NEG, jnp.float32)
            bestc = jnp.full((16,), 0, jnp.int32)
            for g in range(NRO // 16):
                v = cm_v[pl.ds(g * 16, 16)]
                upd = v > bestv
                bestv = jnp.where(upd, v, bestv)
                bestc = jnp.where(upd, g * 16 + lanes, bestc)
            gm = jnp.max(bestv)
            cstar = jnp.min(jnp.where(bestv == gm, bestc, jnp.int32(100000)))
            row8 = lanes < 8
            kv = keys_v[pl.ds(cstar * 8, 16)]
            lstar = jnp.min(jnp.where((kv == gm) & row8, lanes, jnp.int32(16)))
            idx = cstar * 8 + lstar
            score = jnp.where(gm > -1e29, gm, jnp.float32(-jnp.inf))
            lv = labels_v[pl.ds(cstar * 8, 16)]
            lab = jnp.max(jnp.where(lanes == lstar, lv, jnp.int32(-1)))
            box_sel = lanes < 4
            bidx = jnp.where(box_sel, idx * 4 + lanes, 0)
            bv = plsc.load_gather(boxes_v, [bidx], mask=box_sel)
            cx = jnp.max(jnp.where(lanes == 0, bv, NEG))
            cy = jnp.max(jnp.where(lanes == 1, bv, NEG))
            bw = jnp.max(jnp.where(lanes == 2, bv, NEG))
            bh = jnp.max(jnp.where(lanes == 3, bv, NEG))
            x0 = (cx - 0.5 * bw) * ws
            y0 = (cy - 0.5 * bh) * hs
            x1 = (cx + 0.5 * bw) * ws
            y1 = (cy + 0.5 * bh) * hs
            xy = jnp.where(lanes == 0, x0,
                           jnp.where(lanes == 1, y0,
                                     jnp.where(lanes == 2, x1, y1)))
            lane0 = lanes == 0
            plsc.store_scatter(os_v, [jnp.full((16,), k, jnp.int32)],
                               jnp.full((16,), score, jnp.float32), mask=lane0)
            plsc.store_scatter(ol_v, [jnp.full((16,), k, jnp.int32)],
                               jnp.full((16,), lab, jnp.int32), mask=lane0)
            plsc.store_scatter(ob_v, [k * 4 + lanes], xy, mask=box_sel)
            kv2 = jnp.where(lanes == lstar, NEG, kv)
            keys_v[pl.ds(cstar * 8, 16)] = kv2
            rowmax = jnp.max(jnp.where(row8, kv2, NEG))
            plsc.store_scatter(cm_v, [jnp.full((16,), cstar, jnp.int32)],
                               jnp.full((16,), rowmax, jnp.float32),
                               mask=lane0)
            return carry

        lax.fori_loop(0, K, sel_body, jnp.int32(0))
        pltpu.sync_copy(os_v, so_hbm.at[pl.ds(b * KPAD, KPAD)])
        pltpu.sync_copy(ol_v, lo_hbm.at[pl.ds(b * KPAD, KPAD)])
        pltpu.sync_copy(ob_v, bo_hbm.at[pl.ds(b * KPAD * 4, KPAD * 4)])


def kernel(pred_logits, pred_obj, pred_boxes, target_sizes):
    obj_r = jnp.pad(pred_obj, ((0, 0), (0, NP - N))).reshape(B, NRO, 8)

    keys, cmax, labels = pl.pallas_call(
        _tc_body,
        grid=(B // GB,),
        in_specs=[
            pl.BlockSpec((GB, N, 91), lambda b: (b, 0, 0)),
            pl.BlockSpec((GB, NRO, 8), lambda b: (b, 0, 0)),
        ],
        out_specs=[
            pl.BlockSpec((GB, NRO, 8), lambda b: (b, 0, 0)),
            pl.BlockSpec((GB, NRO, 1), lambda b: (b, 0, 0)),
            pl.BlockSpec((GB, NRO, 8), lambda b: (b, 0, 0)),
        ],
        out_shape=[
            jax.ShapeDtypeStruct((B, NRO, 8), jnp.float32),
            jax.ShapeDtypeStruct((B, NRO, 1), jnp.float32),
            jax.ShapeDtypeStruct((B, NRO, 8), jnp.int32),
        ],
        compiler_params=pltpu.CompilerParams(
            dimension_semantics=("arbitrary",)),
    )(pred_logits, obj_r)

    keys_f = keys.reshape(B * NP)
    cm_f = cmax.reshape(B * NRO)
    labels_f = labels.reshape(B * NP)
    boxes_f = pred_boxes.reshape(B * N * 4)
    ts_f = target_sizes.astype(jnp.float32).reshape(B * 2)

    so, lo, bo = _sc_topk(keys_f, cm_f, labels_f, boxes_f, ts_f)
    so = so.reshape(B, KPAD)[:, :K]
    lo = lo.reshape(B, KPAD)[:, :K]
    bo = bo.reshape(B, KPAD, 4)[:, :K]
    res = []
    for b in range(B):
        res.extend([so[b], lo[b], bo[b]])
    return tuple(res)


# speedup vs baseline: 1.0215x; 1.0215x over previous
import functools

import jax
import jax.numpy as jnp
from jax import lax
from jax.experimental import pallas as pl
from jax.experimental.pallas import tpu as pltpu
from jax.experimental.pallas import tpu_sc as plsc

B = 16
N = 5000
NP = 5120
NRI = 625
NRO = 640
NCHUNK = 320
K = 100
KPAD = 104
NEG = -3.0e38


def _tc_body(logits_ref, obj_ref, keys_ref, cmax_ref, labels_ref):
    x = logits_ref[0].reshape(NRI, 8, 91)
    xk = x[..., :90]
    m = jnp.max(xk, axis=-1)
    cls_iota = lax.broadcasted_iota(jnp.int32, (NRI, 8, 90), 2)
    lbl = jnp.min(jnp.where(xk == m[..., None], cls_iota, 1000000), axis=-1)
    o = jnp.exp(-obj_ref[0][:NRI])
    p = jax.nn.sigmoid(m)
    sk = o * p
    su = o * (1.0 - p)
    choose = su > sk
    fs = jnp.where(choose, su, sk)
    flb = jnp.where(choose, 90, lbl)
    keep = fs > 0.05
    gidx = (lax.broadcasted_iota(jnp.int32, (NRI, 8), 0) * 8
            + lax.broadcasted_iota(jnp.int32, (NRI, 8), 1))
    enc = -(1e30 + gidx.astype(jnp.float32) * 1e24)
    key = jnp.where(keep, fs, enc)
    pad_flat = (N
                + lax.broadcasted_iota(jnp.int32, (NRO - NRI, 8), 0) * 8
                + lax.broadcasted_iota(jnp.int32, (NRO - NRI, 8), 1))
    pad_enc = -(1e30 + pad_flat.astype(jnp.float32) * 1e24)
    key_full = jnp.concatenate([key, pad_enc], axis=0)
    keys_ref[0] = key_full
    cmax_ref[0] = jnp.max(key_full, axis=1, keepdims=True)
    labels_ref[0, :NRI] = flb


_sc_mesh = plsc.VectorSubcoreMesh(core_axis_name="c", subcore_axis_name="s")


@functools.partial(
    pl.kernel,
    mesh=_sc_mesh,
    out_type=[
        jax.ShapeDtypeStruct((B * KPAD,), jnp.float32),
        jax.ShapeDtypeStruct((B * KPAD,), jnp.int32),
        jax.ShapeDtypeStruct((B * KPAD * 4,), jnp.float32),
    ],
    scratch_types=[
        pltpu.VMEM((NP + 8,), jnp.float32),
        pltpu.VMEM((NRO,), jnp.float32),
        pltpu.VMEM((NP + 8,), jnp.int32),
        pltpu.VMEM((N * 4,), jnp.float32),
        pltpu.VMEM((32,), jnp.float32),
        pltpu.VMEM((KPAD,), jnp.float32),
        pltpu.VMEM((KPAD,), jnp.int32),
        pltpu.VMEM((KPAD * 4,), jnp.float32),
    ],
    compiler_params=pltpu.CompilerParams(needs_layout_passes=False),
)
def _sc_topk(keys_hbm, cm_hbm, labels_hbm, boxes_hbm, ts_hbm,
             so_hbm, lo_hbm, bo_hbm,
             keys_v, cm_v, labels_v, boxes_v, ts_v, os_v, ol_v, ob_v):
    wid = lax.axis_index("s") * 2 + lax.axis_index("c")

    @pl.when(wid < B)
    def _():
        b = wid
        pltpu.sync_copy(keys_hbm.at[pl.ds(b * NP, NP)],
                        keys_v.at[pl.ds(0, NP)])
        pltpu.sync_copy(cm_hbm.at[pl.ds(b * NRO, NRO)], cm_v)
        pltpu.sync_copy(labels_hbm.at[pl.ds(b * NP, NP)],
                        labels_v.at[pl.ds(0, NP)])
        pltpu.sync_copy(boxes_hbm.at[pl.ds(b * N * 4, N * 4)], boxes_v)
        pltpu.sync_copy(ts_hbm, ts_v)
        lanes = lax.iota(jnp.int32, 16)
        hvec = plsc.load_gather(ts_v, [jnp.full((16,), 2 * b, jnp.int32)])
        wvec = plsc.load_gather(ts_v, [jnp.full((16,), 2 * b + 1, jnp.int32)])
        hs = jnp.max(hvec)
        ws = jnp.max(wvec)
        os_v[pl.ds(KPAD - 16, 16)] = jnp.zeros((16,), jnp.float32)
        ol_v[pl.ds(KPAD - 16, 16)] = jnp.zeros((16,), jnp.int32)
        ob_v[pl.ds(KPAD * 4 - 16, 16)] = jnp.zeros((16,), jnp.float32)

        def sel_body(k, carry):
            bestv = jnp.full((16,), NEG, jnp.float32)
            bestc = jnp.full((16,), 0, jnp.int32)
            for g in range(NRO // 16):
                v = cm_v[pl.ds(g * 16, 16)]
                upd = v > bestv
                bestv = jnp.where(upd, v, bestv)
                bestc = jnp.where(upd, g * 16 + lanes, bestc)
            gm = jnp.max(bestv)
            cstar = jnp.min(jnp.where(bestv == gm, bestc, jnp.int32(100000)))
            row8 = lanes < 8
            kv = keys_v[pl.ds(cstar * 8, 16)]
            lstar = jnp.min(jnp.where((kv == gm) & row8, lanes, jnp.int32(16)))
            idx = cstar * 8 + lstar
            score = jnp.where(gm > -1e29, gm, jnp.float32(-jnp.inf))
            lv = labels_v[pl.ds(cstar * 8, 16)]
            lab = jnp.max(jnp.where(lanes == lstar, lv, jnp.int32(-1)))
            box_sel = lanes < 4
            bidx = jnp.where(box_sel, idx * 4 + lanes, 0)
            bv = plsc.load_gather(boxes_v, [bidx], mask=box_sel)
            cx = jnp.max(jnp.where(lanes == 0, bv, NEG))
            cy = jnp.max(jnp.where(lanes == 1, bv, NEG))
            bw = jnp.max(jnp.where(lanes == 2, bv, NEG))
            bh = jnp.max(jnp.where(lanes == 3, bv, NEG))
            x0 = (cx - 0.5 * bw) * ws
            y0 = (cy - 0.5 * bh) * hs
            x1 = (cx + 0.5 * bw) * ws
            y1 = (cy + 0.5 * bh) * hs
            xy = jnp.where(lanes == 0, x0,
                           jnp.where(lanes == 1, y0,
                                     jnp.where(lanes == 2, x1, y1)))
            lane0 = lanes == 0
            plsc.store_scatter(os_v, [jnp.full((16,), k, jnp.int32)],
                               jnp.full((16,), score, jnp.float32), mask=lane0)
            plsc.store_scatter(ol_v, [jnp.full((16,), k, jnp.int32)],
                               jnp.full((16,), lab, jnp.int32), mask=lane0)
            plsc.store_scatter(ob_v, [k * 4 + lanes], xy, mask=box_sel)
            kv2 = jnp.where(lanes == lstar, NEG, kv)
            keys_v[pl.ds(cstar * 8, 16)] = kv2
            rowmax = jnp.max(jnp.where(row8, kv2, NEG))
            plsc.store_scatter(cm_v, [jnp.full((16,), cstar, jnp.int32)],
                               jnp.full((16,), rowmax, jnp.float32),
                               mask=lane0)
            return carry

        lax.fori_loop(0, K, sel_body, jnp.int32(0))
        pltpu.sync_copy(os_v, so_hbm.at[pl.ds(b * KPAD, KPAD)])
        pltpu.sync_copy(ol_v, lo_hbm.at[pl.ds(b * KPAD, KPAD)])
        pltpu.sync_copy(ob_v, bo_hbm.at[pl.ds(b * KPAD * 4, KPAD * 4)])


def kernel(pred_logits, pred_obj, pred_boxes, target_sizes):
    obj_r = jnp.pad(pred_obj, ((0, 0), (0, NP - N))).reshape(B, NRO, 8)

    keys, cmax, labels = pl.pallas_call(
        _tc_body,
        grid=(B,),
        in_specs=[
            pl.BlockSpec((1, N, 91), lambda b: (b, 0, 0)),
            pl.BlockSpec((1, NRO, 8), lambda b: (b, 0, 0)),
        ],
        out_specs=[
            pl.BlockSpec((1, NRO, 8), lambda b: (b, 0, 0)),
            pl.BlockSpec((1, NRO, 1), lambda b: (b, 0, 0)),
            pl.BlockSpec((1, NRO, 8), lambda b: (b, 0, 0)),
        ],
        out_shape=[
            jax.ShapeDtypeStruct((B, NRO, 8), jnp.float32),
            jax.ShapeDtypeStruct((B, NRO, 1), jnp.float32),
            jax.ShapeDtypeStruct((B, NRO, 8), jnp.int32),
        ],
        compiler_params=pltpu.CompilerParams(
            dimension_semantics=("parallel",)),
    )(pred_logits, obj_r)

    keys_f = keys.reshape(B * NP)
    cm_f = cmax.reshape(B * NRO)
    labels_f = labels.reshape(B * NP)
    boxes_f = pred_boxes.reshape(B * N * 4)
    ts_f = target_sizes.astype(jnp.float32).reshape(B * 2)

    so, lo, bo = _sc_topk(keys_f, cm_f, labels_f, boxes_f, ts_f)
    so = so.reshape(B, KPAD)[:, :K]
    lo = lo.reshape(B, KPAD)[:, :K]
    bo = bo.reshape(B, KPAD, 4)[:, :K]
    res = []
    for b in range(B):
        res.extend([so[b], lo[b], bo[b]])
    return tuple(res)


# speedup vs baseline: 1.0533x; 1.0311x over previous
import functools

import jax
import jax.numpy as jnp
from jax import lax
from jax.experimental import pallas as pl
from jax.experimental.pallas import tpu as pltpu
from jax.experimental.pallas import tpu_sc as plsc

B = 16
N = 5000
NP = 5120
NRI = 625
NRO = 640
NCHUNK = 320
K = 100
KPAD = 104
NEG = -3.0e38


def _tc_body(logits_ref, obj_ref, keys_ref, cmax_ref, labels_ref):
    x = logits_ref[0].reshape(NRI, 8, 91)
    xk = x[..., :90]
    m = jnp.max(xk, axis=-1)
    cls_iota = lax.broadcasted_iota(jnp.int32, (NRI, 8, 90), 2)
    lbl = jnp.min(jnp.where(xk == m[..., None], cls_iota, 1000000), axis=-1)
    o = jnp.exp(-obj_ref[0][:NRI])
    p = jax.nn.sigmoid(m)
    sk = o * p
    su = o * (1.0 - p)
    choose = su > sk
    fs = jnp.where(choose, su, sk)
    flb = jnp.where(choose, 90, lbl)
    keep = fs > 0.05
    gidx = (lax.broadcasted_iota(jnp.int32, (NRI, 8), 0) * 8
            + lax.broadcasted_iota(jnp.int32, (NRI, 8), 1))
    enc = -(1e30 + gidx.astype(jnp.float32) * 1e24)
    key = jnp.where(keep, fs, enc)
    pad_flat = (N
                + lax.broadcasted_iota(jnp.int32, (NRO - NRI, 8), 0) * 8
                + lax.broadcasted_iota(jnp.int32, (NRO - NRI, 8), 1))
    pad_enc = -(1e30 + pad_flat.astype(jnp.float32) * 1e24)
    key_full = jnp.concatenate([key, pad_enc], axis=0)
    keys_ref[0] = key_full
    cmax_ref[0] = jnp.max(key_full, axis=1, keepdims=True)
    labels_ref[0, :NRI] = flb


_sc_mesh = plsc.VectorSubcoreMesh(core_axis_name="c", subcore_axis_name="s")


@functools.partial(
    pl.kernel,
    mesh=_sc_mesh,
    out_type=[
        jax.ShapeDtypeStruct((B * KPAD,), jnp.float32),
        jax.ShapeDtypeStruct((B * KPAD,), jnp.int32),
        jax.ShapeDtypeStruct((B * KPAD * 4,), jnp.float32),
    ],
    scratch_types=[
        pltpu.VMEM((NP + 8,), jnp.float32),
        pltpu.VMEM((NRO,), jnp.float32),
        pltpu.VMEM((NP + 8,), jnp.int32),
        pltpu.VMEM((N * 4,), jnp.float32),
        pltpu.VMEM((32,), jnp.float32),
        pltpu.VMEM((48,), jnp.float32),
        pltpu.SemaphoreType.DMA,
        pltpu.VMEM((KPAD,), jnp.float32),
        pltpu.VMEM((KPAD,), jnp.int32),
        pltpu.VMEM((KPAD * 4,), jnp.float32),
    ],
    compiler_params=pltpu.CompilerParams(needs_layout_passes=False),
)
def _sc_topk(keys_hbm, cm_hbm, labels_hbm, boxes_hbm, ts_hbm,
             so_hbm, lo_hbm, bo_hbm,
             keys_v, cm_v, labels_v, boxes_v, ts_v, gm_v, sem, os_v, ol_v,
             ob_v):
    wid = lax.axis_index("s") * 2 + lax.axis_index("c")

    @pl.when(wid < B)
    def _():
        b = wid
        c1 = pltpu.async_copy(keys_hbm.at[pl.ds(b * NP, NP)],
                              keys_v.at[pl.ds(0, NP)], sem)
        c2 = pltpu.async_copy(cm_hbm.at[pl.ds(b * NRO, NRO)], cm_v, sem)
        c3 = pltpu.async_copy(labels_hbm.at[pl.ds(b * NP, NP)],
                              labels_v.at[pl.ds(0, NP)], sem)
        c4 = pltpu.async_copy(boxes_hbm.at[pl.ds(b * N * 4, N * 4)],
                              boxes_v, sem)
        c5 = pltpu.async_copy(ts_hbm, ts_v, sem)
        c1.wait()
        c2.wait()
        c3.wait()
        c4.wait()
        c5.wait()
        lanes = lax.iota(jnp.int32, 16)
        lane0 = lanes == 0
        hvec = plsc.load_gather(ts_v, [jnp.full((16,), 2 * b, jnp.int32)])
        wvec = plsc.load_gather(ts_v, [jnp.full((16,), 2 * b + 1, jnp.int32)])
        scale_vec = jnp.where((lanes & 1) == 0, wvec, hvec)
        sign_vec = jnp.where(lanes < 2, jnp.float32(-0.5), jnp.float32(0.5))
        gm_v[pl.ds(32, 16)] = jnp.full((16,), NEG, jnp.float32)

        def build_g(g, carry):
            v = cm_v[pl.ds(g * 16, 16)]
            plsc.store_scatter(gm_v, [jnp.full((16,), g, jnp.int32)],
                               jnp.full((16,), jnp.max(v), jnp.float32),
                               mask=lane0)
            return carry

        lax.fori_loop(0, NRO // 16, build_g, jnp.int32(0))
        os_v[pl.ds(KPAD - 16, 16)] = jnp.zeros((16,), jnp.float32)
        ol_v[pl.ds(KPAD - 16, 16)] = jnp.zeros((16,), jnp.int32)
        ob_v[pl.ds(KPAD * 4 - 16, 16)] = jnp.zeros((16,), jnp.float32)

        def sel_body(k, carry):
            bestv = jnp.full((16,), NEG, jnp.float32)
            bestc = jnp.full((16,), 0, jnp.int32)
            for g in range(3):
                v = gm_v[pl.ds(g * 16, 16)]
                upd = v > bestv
                bestv = jnp.where(upd, v, bestv)
                bestc = jnp.where(upd, g * 16 + lanes, bestc)
            gm = jnp.max(bestv)
            gstar = jnp.min(jnp.where(bestv == gm, bestc, jnp.int32(100000)))
            rv = cm_v[pl.ds(gstar * 16, 16)]
            rlane = jnp.min(jnp.where(rv == gm, lanes, jnp.int32(16)))
            cstar = gstar * 16 + rlane
            row8 = lanes < 8
            kv = keys_v[pl.ds(cstar * 8, 16)]
            lstar = jnp.min(jnp.where((kv == gm) & row8, lanes, jnp.int32(16)))
            idx = cstar * 8 + lstar
            score = jnp.where(gm > -1e29, gm, jnp.float32(-jnp.inf))
            lv = labels_v[pl.ds(cstar * 8, 16)]
            lab = jnp.max(jnp.where(lanes == lstar, lv, jnp.int32(-1)))
            box_sel = lanes < 4
            bidx = jnp.where(box_sel, idx * 4 + lanes, 0)
            bv = plsc.load_gather(boxes_v, [bidx], mask=box_sel)
            cx = jnp.max(jnp.where(lanes == 0, bv, NEG))
            cy = jnp.max(jnp.where(lanes == 1, bv, NEG))
            bw = jnp.max(jnp.where(lanes == 2, bv, NEG))
            bh = jnp.max(jnp.where(lanes == 3, bv, NEG))
            center = jnp.where((lanes & 1) == 0, cx, cy)
            whh = jnp.where((lanes & 1) == 0, bw, bh)
            xy = (center + sign_vec * whh) * scale_vec
            plsc.store_scatter(os_v, [jnp.full((16,), k, jnp.int32)],
                               jnp.full((16,), score, jnp.float32), mask=lane0)
            plsc.store_scatter(ol_v, [jnp.full((16,), k, jnp.int32)],
                               jnp.full((16,), lab, jnp.int32), mask=lane0)
            plsc.store_scatter(ob_v, [k * 4 + lanes], xy, mask=box_sel)
            kv2 = jnp.where(lanes == lstar, NEG, kv)
            keys_v[pl.ds(cstar * 8, 16)] = kv2
            rowmax = jnp.max(jnp.where(row8, kv2, NEG))
            plsc.store_scatter(cm_v, [jnp.full((16,), cstar, jnp.int32)],
                               jnp.full((16,), rowmax, jnp.float32),
                               mask=lane0)
            rv2 = jnp.where(lanes == rlane, rowmax, rv)
            plsc.store_scatter(gm_v, [jnp.full((16,), gstar, jnp.int32)],
                               jnp.full((16,), jnp.max(rv2), jnp.float32),
                               mask=lane0)
            return carry

        lax.fori_loop(0, K, sel_body, jnp.int32(0))
        pltpu.sync_copy(os_v, so_hbm.at[pl.ds(b * KPAD, KPAD)])
        pltpu.sync_copy(ol_v, lo_hbm.at[pl.ds(b * KPAD, KPAD)])
        pltpu.sync_copy(ob_v, bo_hbm.at[pl.ds(b * KPAD * 4, KPAD * 4)])


def kernel(pred_logits, pred_obj, pred_boxes, target_sizes):
    obj_r = jnp.pad(pred_obj, ((0, 0), (0, NP - N))).reshape(B, NRO, 8)

    keys, cmax, labels = pl.pallas_call(
        _tc_body,
        grid=(B,),
        in_specs=[
            pl.BlockSpec((1, N, 91), lambda b: (b, 0, 0)),
            pl.BlockSpec((1, NRO, 8), lambda b: (b, 0, 0)),
        ],
        out_specs=[
            pl.BlockSpec((1, NRO, 8), lambda b: (b, 0, 0)),
            pl.BlockSpec((1, NRO, 1), lambda b: (b, 0, 0)),
            pl.BlockSpec((1, NRO, 8), lambda b: (b, 0, 0)),
        ],
        out_shape=[
            jax.ShapeDtypeStruct((B, NRO, 8), jnp.float32),
            jax.ShapeDtypeStruct((B, NRO, 1), jnp.float32),
            jax.ShapeDtypeStruct((B, NRO, 8), jnp.int32),
        ],
        compiler_params=pltpu.CompilerParams(
            dimension_semantics=("parallel",)),
    )(pred_logits, obj_r)

    keys_f = keys.reshape(B * NP)
    cm_f = cmax.reshape(B * NRO)
    labels_f = labels.reshape(B * NP)
    boxes_f = pred_boxes.reshape(B * N * 4)
    ts_f = target_sizes.astype(jnp.float32).reshape(B * 2)

    so, lo, bo = _sc_topk(keys_f, cm_f, labels_f, boxes_f, ts_f)
    so = so.reshape(B, KPAD)[:, :K]
    lo = lo.reshape(B, KPAD)[:, :K]
    bo = bo.reshape(B, KPAD, 4)[:, :K]
    res = []
    for b in range(B):
        res.extend([so[b], lo[b], bo[b]])
    return tuple(res)
